# baseline (speedup 1.0000x reference)
import functools

import jax
import jax.numpy as jnp
from jax import lax
from jax.experimental import pallas as pl
from jax.experimental.pallas import tpu as pltpu
from jax.experimental.pallas import tpu_sc as plsc

B = 16384
D = 64
DP = 2 * D
NC, NS = 2, 16
NW = NC * NS
B_PER_W = B // NW
CHUNK = 128

_pair_t = jax.ShapeDtypeStruct((B, DP), jnp.float32)


@functools.cache
def _build_sc_gather():
    mesh = plsc.VectorSubcoreMesh(core_axis_name="c", subcore_axis_name="s",
                                  num_cores=NC, num_subcores=NS)

    @functools.partial(
        pl.kernel,
        mesh=mesh,
        out_type=(_pair_t, _pair_t, _pair_t, _pair_t),
        scratch_types=[
            pltpu.VMEM((B_PER_W,), jnp.int32),
            pltpu.VMEM((B_PER_W,), jnp.int32),
            pltpu.VMEM((CHUNK, DP), jnp.float32),
            pltpu.VMEM((CHUNK, DP), jnp.float32),
            pltpu.VMEM((CHUNK, DP), jnp.float32),
            pltpu.VMEM((CHUNK, DP), jnp.float32),
            pltpu.SemaphoreType.DMA,
        ],
    )
    def _sc_gather(uhalf_hbm, ihalf_hbm, mfu_hbm, mfi_hbm, mlpu_hbm, mlpi_hbm,
                   mfu_out, mfi_out, mlpu_out, mlpi_out,
                   uidx_v, iidx_v, bmfu, bmfi, bmlpu, bmlpi, sem):
        wid = lax.axis_index("s") * NC + lax.axis_index("c")
        base = wid * B_PER_W
        pltpu.sync_copy(uhalf_hbm.at[pl.ds(base, B_PER_W)], uidx_v)
        pltpu.sync_copy(ihalf_hbm.at[pl.ds(base, B_PER_W)], iidx_v)

        @pl.loop(0, B_PER_W, step=CHUNK)
        def _(off):
            u = uidx_v.at[pl.ds(off, CHUNK)]
            it = iidx_v.at[pl.ds(off, CHUNK)]
            c1 = pltpu.async_copy(mfu_hbm.at[u], bmfu, sem)
            c2 = pltpu.async_copy(mfi_hbm.at[it], bmfi, sem)
            c3 = pltpu.async_copy(mlpu_hbm.at[u], bmlpu, sem)
            c4 = pltpu.async_copy(mlpi_hbm.at[it], bmlpi, sem)
            c1.wait()
            c2.wait()
            c3.wait()
            c4.wait()
            dst = pl.ds(base + off, CHUNK)
            pltpu.sync_copy(bmfu, mfu_out.at[dst])
            pltpu.sync_copy(bmfi, mfi_out.at[dst])
            pltpu.sync_copy(bmlpu, mlpu_out.at[dst])
            pltpu.sync_copy(bmlpi, mlpi_out.at[dst])

    return _sc_gather


def _tc_body(mfu_ref, mfi_ref, mlpu_ref, mlpi_ref, pu_ref, pi_ref,
             w1u_ref, w1i_ref, b1_ref, w2_ref, b2_ref, w3_ref, b3_ref,
             wfmf_ref, wfh_ref, bf_ref, out_ref):
    pu = pu_ref[...]
    pi = pi_ref[...]
    mfu = mfu_ref[:, :D] * (1.0 - pu) + mfu_ref[:, D:] * pu
    mfi = mfi_ref[:, :D] * (1.0 - pi) + mfi_ref[:, D:] * pi
    mlpu = mlpu_ref[:, :D] * (1.0 - pu) + mlpu_ref[:, D:] * pu
    mlpi = mlpi_ref[:, :D] * (1.0 - pi) + mlpi_ref[:, D:] * pi
    mfv = mfu * mfi
    h = jnp.dot(mlpu, w1u_ref[...], preferred_element_type=jnp.float32)
    h = h + jnp.dot(mlpi, w1i_ref[...], preferred_element_type=jnp.float32)
    h = jax.nn.relu(h + b1_ref[...])
    h = jax.nn.relu(jnp.dot(h, w2_ref[...],
                            preferred_element_type=jnp.float32) + b2_ref[...])
    h = jax.nn.relu(jnp.dot(h, w3_ref[...],
                            preferred_element_type=jnp.float32) + b3_ref[...])
    logit = (jnp.sum(mfv * wfmf_ref[...], axis=1, keepdims=True)
             + jnp.sum(h * wfh_ref[...], axis=1, keepdims=True)
             + bf_ref[0, 0])
    out_ref[...] = jax.nn.sigmoid(logit) * 5.0


BLK = 2048


def _tc_compute(mfu, mfi, mlpu, mlpi, pu, pi, w1u, w1i, b1r, w2t, b2r, w3t,
                b3r, wf_mf, wf_h, bf2):
    grid = (B // BLK,)
    pair_spec = pl.BlockSpec((BLK, DP), lambda i: (i, 0))
    par_spec = pl.BlockSpec((BLK, 1), lambda i: (i, 0))

    def full(shape):
        return pl.BlockSpec(shape, lambda i: tuple(0 for _ in shape))

    return pl.pallas_call(
        _tc_body,
        grid=grid,
        in_specs=[
            pair_spec, pair_spec, pair_spec, pair_spec, par_spec, par_spec,
            full(w1u.shape), full(w1i.shape), full(b1r.shape),
            full(w2t.shape), full(b2r.shape),
            full(w3t.shape), full(b3r.shape),
            full(wf_mf.shape), full(wf_h.shape), full(bf2.shape),
        ],
        out_specs=pl.BlockSpec((BLK, 1), lambda i: (i, 0)),
        out_shape=jax.ShapeDtypeStruct((B, 1), jnp.float32),
    )(mfu, mfi, mlpu, mlpi, pu, pi, w1u, w1i, b1r, w2t, b2r, w3t, b3r,
      wf_mf, wf_h, bf2)


def kernel(user_ids, item_ids, mf_user_emb, mf_item_emb, mlp_user_emb,
           mlp_item_emb, W1, b1, W2, b2, W3, b3, Wf, bf):
    user_ids = user_ids.astype(jnp.int32)
    item_ids = item_ids.astype(jnp.int32)
    uhalf = user_ids // 2
    ihalf = item_ids // 2
    pu = (user_ids % 2).astype(jnp.float32).reshape(B, 1)
    pi = (item_ids % 2).astype(jnp.float32).reshape(B, 1)
    mfu2 = mf_user_emb.reshape(-1, DP)
    mfi2 = mf_item_emb.reshape(-1, DP)
    mlpu2 = mlp_user_emb.reshape(-1, DP)
    mlpi2 = mlp_item_emb.reshape(-1, DP)
    mfu, mfi, mlpu, mlpi = _build_sc_gather()(
        uhalf, ihalf, mfu2, mfi2, mlpu2, mlpi2)
    w1u = W1[:, :D].T
    w1i = W1[:, D:].T
    b1r = b1.reshape(1, -1)
    w2t = W2.T
    b2r = b2.reshape(1, -1)
    w3t = W3.T
    b3r = b3.reshape(1, -1)
    wf_mf = Wf[:, :D]
    wf_h = Wf[:, D:]
    bf2 = bf.reshape(1, 1)
    return _tc_compute(mfu, mfi, mlpu, mlpi, pu, pi, w1u, w1i, b1r, w2t, b2r,
                       w3t, b3r, wf_mf, wf_h, bf2)


# speedup vs baseline: 1.5005x; 1.5005x over previous
import functools

import jax
import jax.numpy as jnp
from jax import lax
from jax.experimental import pallas as pl
from jax.experimental.pallas import tpu as pltpu
from jax.experimental.pallas import tpu_sc as plsc

B = 16384
D = 64
NC, NS = 2, 16
NW = NC * NS
B_PER_W = B // NW
CHUNK = 64

_row_t = jax.ShapeDtypeStruct((B, D), jnp.float32)


@functools.cache
def _build_sc_gather():
    mesh = plsc.VectorSubcoreMesh(core_axis_name="c", subcore_axis_name="s",
                                  num_cores=NC, num_subcores=NS)

    @functools.partial(
        pl.kernel,
        mesh=mesh,
        out_type=(_row_t, _row_t, _row_t, _row_t),
        scratch_types=[
            pltpu.SMEM((B_PER_W,), jnp.int32),
            pltpu.SMEM((B_PER_W,), jnp.int32),
            pltpu.VMEM_SHARED((B,), jnp.int32),
            pltpu.VMEM_SHARED((B,), jnp.int32),
            pltpu.VMEM((CHUNK, D), jnp.float32),
            pltpu.VMEM((CHUNK, D), jnp.float32),
            pltpu.VMEM((CHUNK, D), jnp.float32),
            pltpu.VMEM((CHUNK, D), jnp.float32),
            pltpu.SemaphoreType.DMA,
        ],
    )
    def _sc_gather(uidx_hbm, iidx_hbm, mfu_hbm, mfi_hbm, mlpu_hbm, mlpi_hbm,
                   mfu_out, mfi_out, mlpu_out, mlpi_out,
                   uidx_s, iidx_s, ush, ish, bmfu, bmfi, bmlpu, bmlpi, sem):
        wid = lax.axis_index("s") * NC + lax.axis_index("c")
        base = wid * B_PER_W
        sl = pl.ds(base, B_PER_W)
        pltpu.sync_copy(uidx_hbm.at[sl], ush.at[sl])
        pltpu.sync_copy(iidx_hbm.at[sl], ish.at[sl])
        pltpu.sync_copy(ush.at[sl], uidx_s)
        pltpu.sync_copy(ish.at[sl], iidx_s)

        @pl.loop(0, B_PER_W, step=CHUNK)
        def _(off):
            @pl.loop(0, CHUNK)
            def _(r):
                u = jnp.minimum(jnp.maximum(uidx_s[off + r], 0), 999999)
                it = jnp.minimum(jnp.maximum(iidx_s[off + r], 0), 999999)
                pltpu.async_copy(mfu_hbm.at[u], bmfu.at[r], sem)
                pltpu.async_copy(mfi_hbm.at[it], bmfi.at[r], sem)
                pltpu.async_copy(mlpu_hbm.at[u], bmlpu.at[r], sem)
                pltpu.async_copy(mlpi_hbm.at[it], bmlpi.at[r], sem)

            pltpu.make_async_copy(mfu_hbm.at[pl.ds(0, CHUNK)], bmfu,
                                  sem).wait()
            pltpu.make_async_copy(mfi_hbm.at[pl.ds(0, CHUNK)], bmfi,
                                  sem).wait()
            pltpu.make_async_copy(mlpu_hbm.at[pl.ds(0, CHUNK)], bmlpu,
                                  sem).wait()
            pltpu.make_async_copy(mlpi_hbm.at[pl.ds(0, CHUNK)], bmlpi,
                                  sem).wait()
            dst = pl.ds(base + off, CHUNK)
            pltpu.sync_copy(bmfu, mfu_out.at[dst])
            pltpu.sync_copy(bmfi, mfi_out.at[dst])
            pltpu.sync_copy(bmlpu, mlpu_out.at[dst])
            pltpu.sync_copy(bmlpi, mlpi_out.at[dst])

    return _sc_gather


def _tc_body(mfu_ref, mfi_ref, mlpu_ref, mlpi_ref, w1u_ref, w1i_ref, b1_ref,
             w2_ref, b2_ref, w3_ref, b3_ref, wfmf_ref, wfh_ref, bf_ref,
             out_ref):
    mfv = mfu_ref[...] * mfi_ref[...]
    h = jnp.dot(mlpu_ref[...], w1u_ref[...],
                preferred_element_type=jnp.float32)
    h = h + jnp.dot(mlpi_ref[...], w1i_ref[...],
                    preferred_element_type=jnp.float32)
    h = jax.nn.relu(h + b1_ref[...])
    h = jax.nn.relu(jnp.dot(h, w2_ref[...],
                            preferred_element_type=jnp.float32) + b2_ref[...])
    h = jax.nn.relu(jnp.dot(h, w3_ref[...],
                            preferred_element_type=jnp.float32) + b3_ref[...])
    logit = (jnp.sum(mfv * wfmf_ref[...], axis=1, keepdims=True)
             + jnp.sum(h * wfh_ref[...], axis=1, keepdims=True)
             + bf_ref[0, 0])
    out_ref[...] = jax.nn.sigmoid(logit) * 5.0


BLK = 2048


def _tc_compute(mfu, mfi, mlpu, mlpi, w1u, w1i, b1r, w2t, b2r, w3t, b3r,
                wf_mf, wf_h, bf2):
    grid = (B // BLK,)
    row_spec = pl.BlockSpec((BLK, D), lambda i: (i, 0))

    def full(shape):
        return pl.BlockSpec(shape, lambda i: tuple(0 for _ in shape))

    return pl.pallas_call(
        _tc_body,
        grid=grid,
        in_specs=[
            row_spec, row_spec, row_spec, row_spec,
            full(w1u.shape), full(w1i.shape), full(b1r.shape),
            full(w2t.shape), full(b2r.shape),
            full(w3t.shape), full(b3r.shape),
            full(wf_mf.shape), full(wf_h.shape), full(bf2.shape),
        ],
        out_specs=pl.BlockSpec((BLK, 1), lambda i: (i, 0)),
        out_shape=jax.ShapeDtypeStruct((B, 1), jnp.float32),
    )(mfu, mfi, mlpu, mlpi, w1u, w1i, b1r, w2t, b2r, w3t, b3r,
      wf_mf, wf_h, bf2)


def kernel(user_ids, item_ids, mf_user_emb, mf_item_emb, mlp_user_emb,
           mlp_item_emb, W1, b1, W2, b2, W3, b3, Wf, bf):
    user_ids = user_ids.astype(jnp.int32)
    item_ids = item_ids.astype(jnp.int32)
    mfu, mfi, mlpu, mlpi = _build_sc_gather()(
        user_ids, item_ids, mf_user_emb, mf_item_emb, mlp_user_emb,
        mlp_item_emb)
    w1u = W1[:, :D].T
    w1i = W1[:, D:].T
    b1r = b1.reshape(1, -1)
    w2t = W2.T
    b2r = b2.reshape(1, -1)
    w3t = W3.T
    b3r = b3.reshape(1, -1)
    wf_mf = Wf[:, :D]
    wf_h = Wf[:, D:]
    bf2 = bf.reshape(1, 1)
    return _tc_compute(mfu, mfi, mlpu, mlpi, w1u, w1i, b1r, w2t, b2r, w3t,
                       b3r, wf_mf, wf_h, bf2)
